# Initial kernel scaffold; baseline (speedup 1.0000x reference)
import functools

import jax
import jax.numpy as jnp
from jax import lax
from jax.experimental import pallas as pl
from jax.experimental.pallas import tpu as pltpu
from jax.experimental.pallas import tpu_sc as plsc

_MAX_SLIP = 0.4
_MAX_GUESS = 0.4
_LN2 = 0.6931471805599453

_B = 16384
_H = 128
_NC = 2
_NS = 16
_NW = _NC * _NS
_BPW = _B // _NW
_CH = 128
_NCHUNK = _BPW // _CH
_L = 16


def _vlog(x):
    bits = plsc.bitcast(x, jnp.int32)
    e = ((bits >> 23) & 0xFF) - 127
    m = plsc.bitcast((bits & 0x7FFFFF) | (127 << 23), jnp.float32)
    big = m > 1.4142135
    m = jnp.where(big, m * 0.5, m)
    ef = jnp.where(big, e + 1, e).astype(jnp.float32)
    z = (m - 1.0) / (m + 1.0)
    z2 = z * z
    p = z * (2.0 + z2 * (0.6666667 + z2 * (0.4 + z2 * (0.2857143 + z2 * 0.2222222))))
    return ef * _LN2 + p


def _sigmoid_scaled(x, scale):
    return scale / (1.0 + jnp.exp(-x))


def _body(user_ref, item_ref, knowledge_ref, theta_ref, slip_ref, guess_ref,
          out_ref, uidx, iidx, tbuf0, tbuf1, kbuf0, kbuf1, sraw, graw,
          cvals, obuf, semt0, semt1, semk0, semk1, semsg):
    wid = lax.axis_index("s") * _NC + lax.axis_index("c")
    rowbase = wid * _NCHUNK
    ebase = wid * _BPW

    pltpu.sync_copy(user_ref.at[pl.ds(rowbase, _NCHUNK)], uidx)
    pltpu.sync_copy(item_ref.at[pl.ds(rowbase, _NCHUNK)], iidx)

    sg_copies = []
    for k in range(_NCHUNK):
        c1 = pltpu.make_async_copy(slip_ref.at[iidx.at[k]], sraw.at[k], semsg)
        c1.start()
        c2 = pltpu.make_async_copy(guess_ref.at[iidx.at[k]], graw.at[k], semsg)
        c2.start()
        sg_copies += [c1, c2]

    tbuf = [tbuf0, tbuf1]
    kbuf = [kbuf0, kbuf1]
    semt = [semt0, semt1]
    semk = [semk0, semk1]

    def chunk_copies(k):
        b = k % 2
        tc = pltpu.make_async_copy(theta_ref.at[uidx.at[k]], tbuf[b], semt[b])
        kc = pltpu.make_async_copy(
            knowledge_ref.at[pl.ds(ebase + k * _CH, _CH)], kbuf[b], semk[b])
        return tc, kc

    started = [chunk_copies(k) for k in range(_NCHUNK)]
    for c in started[0]:
        c.start()
    for c in started[1]:
        c.start()

    lane = lax.iota(jnp.int32, _L)

    for k in range(_NCHUNK):
        b = k % 2
        for c in started[k]:
            c.wait()
        tb, kb = tbuf[b], kbuf[b]

        def group(g, _, tb=tb, kb=kb, k=k):
            res = jnp.zeros((_L,), jnp.int32)
            for e in range(_L):
                row = g * _L + e
                tot = jnp.zeros((_L,), jnp.int32)
                for h in range(_H // _L):
                    t = tb[row, pl.ds(h * _L, _L)]
                    kn = kb[row, pl.ds(h * _L, _L)]
                    m = (kn == 1.0) & (t <= 0.0)
                    tot = tot + plsc.all_reduce_population_count(m)
                res = jnp.where(lane == e, tot, res)
            cvals[k, pl.ds(g * _L, _L)] = res
            return 0

        lax.fori_loop(0, _CH // _L, group, 0)

        if k + 2 < _NCHUNK:
            for c in started[k + 2]:
                c.start()

    for c in sg_copies:
        c.wait()

    for k in range(_NCHUNK):
        def pgroup(j, _, k=k):
            c = cvals[k, pl.ds(j * _L, _L)]
            sr = sraw[k, pl.ds(j * _L, _L)]
            gr = graw[k, pl.ds(j * _L, _L)]
            s = _sigmoid_scaled(sr, _MAX_SLIP)
            g = _sigmoid_scaled(gr, _MAX_GUESS)
            nbits = (127 - c) << 23
            n = jnp.where(c < 127, plsc.bitcast(nbits, jnp.float32), 0.0)
            r = jnp.exp(n * _vlog(1.0 - s) + (1.0 - n) * _vlog(g))
            obuf[k, pl.ds(j * _L, _L)] = r
            return 0

        lax.fori_loop(0, _CH // _L, pgroup, 0)

    pltpu.sync_copy(obuf, out_ref.at[pl.ds(rowbase, _NCHUNK)])


_sc_kernel = functools.partial(
    pl.kernel,
    mesh=plsc.VectorSubcoreMesh(core_axis_name="c", subcore_axis_name="s"),
    out_type=jax.ShapeDtypeStruct((_B // _H, _H), jnp.float32),
    scratch_types=[
        pltpu.VMEM((_NCHUNK, _CH), jnp.int32),
        pltpu.VMEM((_NCHUNK, _CH), jnp.int32),
        pltpu.VMEM((_CH, _H), jnp.float32),
        pltpu.VMEM((_CH, _H), jnp.float32),
        pltpu.VMEM((_CH, _H), jnp.float32),
        pltpu.VMEM((_CH, _H), jnp.float32),
        pltpu.VMEM((_NCHUNK, _CH), jnp.float32),
        pltpu.VMEM((_NCHUNK, _CH), jnp.float32),
        pltpu.VMEM((_NCHUNK, _CH), jnp.int32),
        pltpu.VMEM((_NCHUNK, _CH), jnp.float32),
        pltpu.SemaphoreType.DMA,
        pltpu.SemaphoreType.DMA,
        pltpu.SemaphoreType.DMA,
        pltpu.SemaphoreType.DMA,
        pltpu.SemaphoreType.DMA,
    ],
)(_body)


def kernel(user, item, knowledge, theta_table, slip_table, guess_table):
    user2d = user.astype(jnp.int32).reshape(_B // _H, _H)
    item2d = item.astype(jnp.int32).reshape(_B // _H, _H)
    out2d = _sc_kernel(user2d, item2d, knowledge, theta_table,
                       slip_table.reshape(-1), guess_table.reshape(-1))
    return out2d.reshape(-1)


# speedup vs baseline: 1.2058x; 1.2058x over previous
import functools

import jax
import jax.numpy as jnp
from jax import lax
from jax.experimental import pallas as pl
from jax.experimental.pallas import tpu as pltpu
from jax.experimental.pallas import tpu_sc as plsc

_MAX_SLIP = 0.4
_MAX_GUESS = 0.4
_LN2 = 0.6931471805599453

_B = 16384
_H = 128
_NC = 2
_NS = 16
_NW = _NC * _NS
_BPW = _B // _NW
_CH = 128
_NCHUNK = _BPW // _CH
_L = 16


def _vlog(x):
    bits = plsc.bitcast(x, jnp.int32)
    e = ((bits >> 23) & 0xFF) - 127
    m = plsc.bitcast((bits & 0x7FFFFF) | (127 << 23), jnp.float32)
    big = m > 1.4142135
    m = jnp.where(big, m * 0.5, m)
    ef = jnp.where(big, e + 1, e).astype(jnp.float32)
    z = (m - 1.0) / (m + 1.0)
    z2 = z * z
    p = z * (2.0 + z2 * (0.6666667 + z2 * (0.4 + z2 * (0.2857143 + z2 * 0.2222222))))
    return ef * _LN2 + p


def _sigmoid_scaled(x, scale):
    return scale / (1.0 + jnp.exp(-x))


def _body(user_ref, item_ref, knowledge_ref, theta_ref, slip_ref, guess_ref,
          out_ref, uidx, iidx, tbuf0, tbuf1, kbuf0, kbuf1, sraw, graw,
          cvals, obuf, semt0, semt1, semk0, semk1, semsg):
    wid = lax.axis_index("s") * _NC + lax.axis_index("c")
    rowbase = wid * _NCHUNK
    ebase = wid * _BPW

    pltpu.sync_copy(user_ref.at[pl.ds(rowbase, _NCHUNK)], uidx)
    pltpu.sync_copy(item_ref.at[pl.ds(rowbase, _NCHUNK)], iidx)

    sg_copies = []
    for k in range(_NCHUNK):
        c1 = pltpu.make_async_copy(slip_ref.at[iidx.at[k]], sraw.at[k], semsg)
        c1.start()
        c2 = pltpu.make_async_copy(guess_ref.at[iidx.at[k]], graw.at[k], semsg)
        c2.start()
        sg_copies += [c1, c2]

    tbuf = [tbuf0, tbuf1]
    kbuf = [kbuf0, kbuf1]
    semt = [semt0, semt1]
    semk = [semk0, semk1]

    def chunk_copies(k):
        b = k % 2
        tc = pltpu.make_async_copy(theta_ref.at[uidx.at[k]], tbuf[b], semt[b])
        kc = pltpu.make_async_copy(
            knowledge_ref.at[pl.ds(ebase + k * _CH, _CH)], kbuf[b], semk[b])
        return tc, kc

    started = [chunk_copies(k) for k in range(_NCHUNK)]
    for c in started[0]:
        c.start()
    for c in started[1]:
        c.start()

    lane = lax.iota(jnp.int32, _L)

    for k in range(_NCHUNK):
        b = k % 2
        for c in started[k]:
            c.wait()
        tb, kb = tbuf[b], kbuf[b]

        def group(g, _, tb=tb, kb=kb, k=k):
            res = jnp.zeros((_L,), jnp.int32)
            for e in range(_L):
                row = g * _L + e
                tot = jnp.zeros((_L,), jnp.int32)
                for h in range(_H // _L):
                    t = tb[row, pl.ds(h * _L, _L)]
                    kn = kb[row, pl.ds(h * _L, _L)]
                    m = (kn == 1.0) & (t <= 0.0)
                    tot = tot + plsc.all_reduce_population_count(m)
                res = jnp.where(lane == e, tot, res)
            cvals[k, pl.ds(g * _L, _L)] = res
            return 0

        lax.fori_loop(0, _CH // _L, group, 0)

        if k + 2 < _NCHUNK:
            for c in started[k + 2]:
                c.start()

    for c in sg_copies:
        c.wait()

    for k in range(_NCHUNK):
        def pgroup(j, _, k=k):
            c = cvals[k, pl.ds(j * _L, _L)]
            sr = sraw[k, pl.ds(j * _L, _L)]
            gr = graw[k, pl.ds(j * _L, _L)]
            s = _sigmoid_scaled(sr, _MAX_SLIP)
            g = _sigmoid_scaled(gr, _MAX_GUESS)
            nbits = (127 - c) << 23
            n = jnp.where(c < 127, plsc.bitcast(nbits, jnp.float32), 0.0)
            r = jnp.exp(n * _vlog(1.0 - s) + (1.0 - n) * _vlog(g))
            obuf[k, pl.ds(j * _L, _L)] = r
            return 0

        lax.fori_loop(0, _CH // _L, pgroup, 0)

    pltpu.sync_copy(obuf, out_ref.at[pl.ds(rowbase, _NCHUNK)])


_sc_kernel = functools.partial(
    pl.kernel,
    mesh=plsc.VectorSubcoreMesh(core_axis_name="c", subcore_axis_name="s"),
    out_type=jax.ShapeDtypeStruct((_B // _H, _H), jnp.float32),
    compiler_params=pltpu.CompilerParams(needs_layout_passes=False),
    scratch_types=[
        pltpu.VMEM((_NCHUNK, _CH), jnp.int32),
        pltpu.VMEM((_NCHUNK, _CH), jnp.int32),
        pltpu.VMEM((_CH, _H), jnp.float32),
        pltpu.VMEM((_CH, _H), jnp.float32),
        pltpu.VMEM((_CH, _H), jnp.float32),
        pltpu.VMEM((_CH, _H), jnp.float32),
        pltpu.VMEM((_NCHUNK, _CH), jnp.float32),
        pltpu.VMEM((_NCHUNK, _CH), jnp.float32),
        pltpu.VMEM((_NCHUNK, _CH), jnp.int32),
        pltpu.VMEM((_NCHUNK, _CH), jnp.float32),
        pltpu.SemaphoreType.DMA,
        pltpu.SemaphoreType.DMA,
        pltpu.SemaphoreType.DMA,
        pltpu.SemaphoreType.DMA,
        pltpu.SemaphoreType.DMA,
    ],
)(_body)


def kernel(user, item, knowledge, theta_table, slip_table, guess_table):
    user2d = user.astype(jnp.int32).reshape(_B // _H, _H)
    item2d = item.astype(jnp.int32).reshape(_B // _H, _H)
    out2d = _sc_kernel(user2d, item2d, knowledge, theta_table,
                       slip_table.reshape(-1), guess_table.reshape(-1))
    return out2d.reshape(-1)


# speedup vs baseline: 1.4879x; 1.2340x over previous
import functools

import jax
import jax.numpy as jnp
from jax import lax
from jax.experimental import pallas as pl
from jax.experimental.pallas import tpu as pltpu
from jax.experimental.pallas import tpu_sc as plsc

_MAX_SLIP = 0.4
_MAX_GUESS = 0.4
_LN2 = 0.6931471805599453

_B = 16384
_H = 128
_NC = 2
_NS = 16
_NW = _NC * _NS
_BPW = _B // _NW
_CH = 128
_NCHUNK = _BPW // _CH
_L = 16


def _vlog(x):
    bits = plsc.bitcast(x, jnp.int32)
    e = ((bits >> 23) & 0xFF) - 127
    m = plsc.bitcast((bits & 0x7FFFFF) | (127 << 23), jnp.float32)
    big = m > 1.4142135
    m = jnp.where(big, m * 0.5, m)
    ef = jnp.where(big, e + 1, e).astype(jnp.float32)
    z = (m - 1.0) / (m + 1.0)
    z2 = z * z
    p = z * (2.0 + z2 * (0.6666667 + z2 * (0.4 + z2 * (0.2857143 + z2 * 0.2222222))))
    return ef * _LN2 + p


def _sigmoid_scaled(x, scale):
    return scale / (1.0 + jnp.exp(-x))


def _body(user_ref, item_ref, knowledge_ref, theta_ref, slip_ref, guess_ref,
          out_ref, uidx, iidx, tbuf0, tbuf1, kbuf0, kbuf1, sraw, graw,
          cvals, obuf, semt0, semt1, semk0, semk1, semsg):
    wid = lax.axis_index("s") * _NC + lax.axis_index("c")
    rowbase = wid * _NCHUNK
    ebase = wid * _BPW

    pltpu.sync_copy(user_ref.at[pl.ds(rowbase, _NCHUNK)], uidx)
    pltpu.sync_copy(item_ref.at[pl.ds(rowbase, _NCHUNK)], iidx)

    sg_copies = []
    for k in range(_NCHUNK):
        c1 = pltpu.make_async_copy(slip_ref.at[iidx.at[k]], sraw.at[k], semsg)
        c1.start()
        c2 = pltpu.make_async_copy(guess_ref.at[iidx.at[k]], graw.at[k], semsg)
        c2.start()
        sg_copies += [c1, c2]

    tbuf = [tbuf0, tbuf1]
    kbuf = [kbuf0, kbuf1]
    semt = [semt0, semt1]
    semk = [semk0, semk1]

    def chunk_copies(k):
        b = k % 2
        tc = pltpu.make_async_copy(theta_ref.at[uidx.at[k]], tbuf[b], semt[b])
        kc = pltpu.make_async_copy(
            knowledge_ref.at[pl.ds(ebase + k * _CH, _CH)], kbuf[b], semk[b])
        return tc, kc

    started = [chunk_copies(k) for k in range(_NCHUNK)]
    for c in started[0]:
        c.start()
    for c in started[1]:
        c.start()

    lane = lax.iota(jnp.int32, _L)

    for k in range(_NCHUNK):
        b = k % 2
        with jax.named_scope(f"wait{k}"):
            for c in started[k]:
                c.wait()
        tb, kb = tbuf[b], kbuf[b]

        def group(g, _, tb=tb, kb=kb, k=k):
            rows = g * _L + lane

            def hblock(hb, carry):
                cols = carry[0]
                accs = list(carry[1:])
                for h in range(32):
                    t = plsc.load_gather(tb, [rows, cols])
                    kn = plsc.load_gather(kb, [rows, cols])
                    j = h % 4
                    accs[j] = jnp.where(t <= 0.0, accs[j] + kn, accs[j])
                    cols = (cols + 1) & (_H - 1)
                return (cols, *accs)

            z = jnp.zeros((_L,), jnp.float32)
            carry = lax.fori_loop(0, _H // 32, hblock, (lane, z, z, z, z))
            cvals[k, pl.ds(g * _L, _L)] = (carry[1] + carry[2]) + (carry[3] + carry[4])
            return 0

        with jax.named_scope(f"comp{k}"):
            lax.fori_loop(0, _CH // _L, group, 0)

        if k + 2 < _NCHUNK:
            for c in started[k + 2]:
                c.start()

    for c in sg_copies:
        c.wait()

    for k in range(_NCHUNK):
        def pgroup(j, _, k=k):
            c = cvals[k, pl.ds(j * _L, _L)].astype(jnp.int32)
            sr = sraw[k, pl.ds(j * _L, _L)]
            gr = graw[k, pl.ds(j * _L, _L)]
            s = _sigmoid_scaled(sr, _MAX_SLIP)
            g = _sigmoid_scaled(gr, _MAX_GUESS)
            nbits = (127 - c) << 23
            n = jnp.where(c < 127, plsc.bitcast(nbits, jnp.float32), 0.0)
            r = jnp.exp(n * _vlog(1.0 - s) + (1.0 - n) * _vlog(g))
            obuf[k, pl.ds(j * _L, _L)] = r
            return 0

        lax.fori_loop(0, _CH // _L, pgroup, 0)

    pltpu.sync_copy(obuf, out_ref.at[pl.ds(rowbase, _NCHUNK)])


_sc_kernel = functools.partial(
    pl.kernel,
    mesh=plsc.VectorSubcoreMesh(core_axis_name="c", subcore_axis_name="s"),
    out_type=jax.ShapeDtypeStruct((_B // _H, _H), jnp.float32),
    compiler_params=pltpu.CompilerParams(needs_layout_passes=False),
    scratch_types=[
        pltpu.VMEM((_NCHUNK, _CH), jnp.int32),
        pltpu.VMEM((_NCHUNK, _CH), jnp.int32),
        pltpu.VMEM((_CH, _H), jnp.float32),
        pltpu.VMEM((_CH, _H), jnp.float32),
        pltpu.VMEM((_CH, _H), jnp.float32),
        pltpu.VMEM((_CH, _H), jnp.float32),
        pltpu.VMEM((_NCHUNK, _CH), jnp.float32),
        pltpu.VMEM((_NCHUNK, _CH), jnp.float32),
        pltpu.VMEM((_NCHUNK, _CH), jnp.float32),
        pltpu.VMEM((_NCHUNK, _CH), jnp.float32),
        pltpu.SemaphoreType.DMA,
        pltpu.SemaphoreType.DMA,
        pltpu.SemaphoreType.DMA,
        pltpu.SemaphoreType.DMA,
        pltpu.SemaphoreType.DMA,
    ],
)(_body)


def kernel(user, item, knowledge, theta_table, slip_table, guess_table):
    user2d = user.astype(jnp.int32).reshape(_B // _H, _H)
    item2d = item.astype(jnp.int32).reshape(_B // _H, _H)
    out2d = _sc_kernel(user2d, item2d, knowledge, theta_table,
                       slip_table.reshape(-1), guess_table.reshape(-1))
    return out2d.reshape(-1)
